# baseline (speedup 1.0000x reference)
import functools

import jax
import jax.numpy as jnp
from jax import lax
from jax.experimental import pallas as pl
from jax.experimental.pallas import tpu as pltpu
from jax.experimental.pallas import tpu_sc as plsc

_B, _S, _D = 4, 8192, 1024
_SD = _S * _D
_NW = 32
_PW = _SD // _NW
_CW = 32768
_NCHUNK = _PW // _CW
_L = 16


def _sc_body(x_hbm, t_hbm, o_hbm, t_buf, x_buf):
    wid = lax.axis_index("s") * 2 + lax.axis_index("c")
    base = wid * _PW
    for chunk in range(_NCHUNK):
        t_off = base + chunk * _CW
        pltpu.sync_copy(t_hbm.at[pl.ds(t_off, _CW)], t_buf)
        for b in range(_B):
            x_off = b * _SD + t_off
            pltpu.sync_copy(x_hbm.at[pl.ds(x_off, _CW)], x_buf)

            def _add(i, carry):
                sl = pl.ds(i * _L, _L)
                x_buf[sl] = x_buf[sl] + t_buf[sl]
                return carry

            lax.fori_loop(0, _CW // _L, _add, 0)
            pltpu.sync_copy(x_buf, o_hbm.at[pl.ds(x_off, _CW)])


def kernel(x, table):
    mesh = plsc.VectorSubcoreMesh(core_axis_name="c", subcore_axis_name="s")
    k = functools.partial(
        pl.kernel,
        mesh=mesh,
        out_type=jax.ShapeDtypeStruct((_B * _SD,), jnp.float32),
        scratch_types=[
            pltpu.VMEM((_CW,), jnp.float32),
            pltpu.VMEM((_CW,), jnp.float32),
        ],
    )(_sc_body)
    out = k(x.reshape(-1), table.reshape(-1))
    return out.reshape(_B, _S, _D)


# speedup vs baseline: 1.5370x; 1.5370x over previous
import functools

import jax
import jax.numpy as jnp
from jax import lax
from jax.experimental import pallas as pl
from jax.experimental.pallas import tpu as pltpu
from jax.experimental.pallas import tpu_sc as plsc

_B, _S, _D = 4, 8192, 1024
_C = 512
_TROWS = _S * _D // _C
_XROWS = _B * _TROWS
_NW = 32
_RW = _TROWS // _NW
_RC = 16
_NCH = _RW // _RC
_KV = _C // 16


def _sc_body(x_hbm, t_hbm, o_hbm, xA, xB, tA, tB, siA, siB, stA, stB,
             soA, soB):
    cid = lax.axis_index("c")
    sid = lax.axis_index("s")
    wid = sid * 2 + cid
    trow0 = wid * _RW

    def in_copy(c, xbuf, b, sem):
        row = b * _TROWS + trow0 + c * _RC
        return pltpu.make_async_copy(
            x_hbm.at[pl.ds(row, _RC)], xbuf.at[b], sem)

    def out_copy(c, xbuf, b, sem):
        row = b * _TROWS + trow0 + c * _RC
        return pltpu.make_async_copy(
            xbuf.at[b], o_hbm.at[pl.ds(row, _RC)], sem)

    def t_copy(c, tbuf, sem):
        return pltpu.make_async_copy(
            t_hbm.at[pl.ds(trow0 + c * _RC, _RC)], tbuf, sem)

    def start_chunk(c, xbuf, tbuf, sem_in, sem_t):
        t_copy(c, tbuf, sem_t).start()
        for b in range(_B):
            in_copy(c, xbuf, b, sem_in).start()

    def wait_chunk_in(c, xbuf, tbuf, sem_in, sem_t):
        t_copy(c, tbuf, sem_t).wait()
        for b in range(_B):
            in_copy(c, xbuf, b, sem_in).wait()

    def compute(xbuf, tbuf):
        def row_body(i, carry):
            tv = [tbuf[i, pl.ds(k * 16, 16)] for k in range(_KV)]
            for b in range(_B):
                for k in range(_KV):
                    sl = (b, i, pl.ds(k * 16, 16))
                    xbuf[sl] = xbuf[sl] + tv[k]
            return carry

        lax.fori_loop(0, _RC, row_body, 0)

    start_chunk(0, xA, tA, siA, stA)
    start_chunk(1, xB, tB, siB, stB)

    def iteration(g, carry):
        c0 = 2 * g
        c1 = c0 + 1
        wait_chunk_in(c0, xA, tA, siA, stA)
        compute(xA, tA)
        for b in range(_B):
            out_copy(c0, xA, b, soA).start()
        wait_chunk_in(c1, xB, tB, siB, stB)
        compute(xB, tB)
        for b in range(_B):
            out_copy(c1, xB, b, soB).start()
        for b in range(_B):
            out_copy(c0, xA, b, soA).wait()

        @pl.when(c0 + 2 < _NCH)
        def _():
            start_chunk(c0 + 2, xA, tA, siA, stA)

        for b in range(_B):
            out_copy(c1, xB, b, soB).wait()

        @pl.when(c1 + 2 < _NCH)
        def _():
            start_chunk(c1 + 2, xB, tB, siB, stB)

        return carry

    lax.fori_loop(0, _NCH // 2, iteration, 0)


def kernel(x, table):
    mesh = plsc.VectorSubcoreMesh(core_axis_name="c", subcore_axis_name="s")
    k = functools.partial(
        pl.kernel,
        mesh=mesh,
        out_type=jax.ShapeDtypeStruct((_XROWS, _C), jnp.float32),
        scratch_types=[
            pltpu.VMEM((_B, _RC, _C), jnp.float32),
            pltpu.VMEM((_B, _RC, _C), jnp.float32),
            pltpu.VMEM((_RC, _C), jnp.float32),
            pltpu.VMEM((_RC, _C), jnp.float32),
            pltpu.SemaphoreType.DMA,
            pltpu.SemaphoreType.DMA,
            pltpu.SemaphoreType.DMA,
            pltpu.SemaphoreType.DMA,
            pltpu.SemaphoreType.DMA,
            pltpu.SemaphoreType.DMA,
        ],
    )(_sc_body)
    out = k(x.reshape(_XROWS, _C), table.reshape(_TROWS, _C))
    return out.reshape(_B, _S, _D)


# speedup vs baseline: 1.6780x; 1.0917x over previous
import functools

import jax
import jax.numpy as jnp
from jax import lax
from jax.experimental import pallas as pl
from jax.experimental.pallas import tpu as pltpu
from jax.experimental.pallas import tpu_sc as plsc

_B, _S, _D = 4, 8192, 1024
_SD = _S * _D
_NW = 32
_PW = _SD // _NW
_CHW = 8192
_NCH = _PW // _CHW


def _sc_body(x_hbm, t_hbm, o_hbm, xA, xB, tA, tB, siA, siB, stA, stB,
             soA, soB):
    cid = lax.axis_index("c")
    sid = lax.axis_index("s")
    wid = sid * 2 + cid
    tw0 = wid * _PW

    def in_copy(c, xbuf, b, sem):
        off = b * _SD + tw0 + c * _CHW
        return pltpu.make_async_copy(
            x_hbm.at[pl.ds(off, _CHW)], xbuf.at[b], sem)

    def out_copy(c, xbuf, b, sem):
        off = b * _SD + tw0 + c * _CHW
        return pltpu.make_async_copy(
            xbuf.at[b], o_hbm.at[pl.ds(off, _CHW)], sem)

    def t_copy(c, tbuf, sem):
        return pltpu.make_async_copy(
            t_hbm.at[pl.ds(tw0 + c * _CHW, _CHW)], tbuf, sem)

    def start_chunk(c, xbuf, tbuf, sem_in, sem_t):
        t_copy(c, tbuf, sem_t).start()
        for b in range(_B):
            in_copy(c, xbuf, b, sem_in).start()

    def wait_chunk_in(c, xbuf, tbuf, sem_in, sem_t):
        t_copy(c, tbuf, sem_t).wait()
        for b in range(_B):
            in_copy(c, xbuf, b, sem_in).wait()

    def compute(xbuf, tbuf):
        @plsc.parallel_loop(0, _CHW // 16, unroll=4)
        def _(j):
            sl = pl.ds(j * 16, 16)
            tv = tbuf[sl]
            for b in range(_B):
                xbuf[b, sl] = xbuf[b, sl] + tv

    start_chunk(0, xA, tA, siA, stA)
    start_chunk(1, xB, tB, siB, stB)

    def iteration(g, carry):
        c0 = 2 * g
        c1 = c0 + 1
        wait_chunk_in(c0, xA, tA, siA, stA)
        compute(xA, tA)
        for b in range(_B):
            out_copy(c0, xA, b, soA).start()
        wait_chunk_in(c1, xB, tB, siB, stB)
        compute(xB, tB)
        for b in range(_B):
            out_copy(c1, xB, b, soB).start()
        for b in range(_B):
            out_copy(c0, xA, b, soA).wait()

        @pl.when(c0 + 2 < _NCH)
        def _():
            start_chunk(c0 + 2, xA, tA, siA, stA)

        for b in range(_B):
            out_copy(c1, xB, b, soB).wait()

        @pl.when(c1 + 2 < _NCH)
        def _():
            start_chunk(c1 + 2, xB, tB, siB, stB)

        return carry

    lax.fori_loop(0, _NCH // 2, iteration, 0)


def kernel(x, table):
    mesh = plsc.VectorSubcoreMesh(core_axis_name="c", subcore_axis_name="s")
    k = functools.partial(
        pl.kernel,
        mesh=mesh,
        out_type=jax.ShapeDtypeStruct((_B * _SD,), jnp.float32),
        scratch_types=[
            pltpu.VMEM((_B, _CHW), jnp.float32),
            pltpu.VMEM((_B, _CHW), jnp.float32),
            pltpu.VMEM((_CHW,), jnp.float32),
            pltpu.VMEM((_CHW,), jnp.float32),
            pltpu.SemaphoreType.DMA,
            pltpu.SemaphoreType.DMA,
            pltpu.SemaphoreType.DMA,
            pltpu.SemaphoreType.DMA,
            pltpu.SemaphoreType.DMA,
            pltpu.SemaphoreType.DMA,
        ],
    )(_sc_body)
    out = k(x.reshape(-1), table.reshape(-1))
    return out.reshape(_B, _S, _D)


# speedup vs baseline: 3.3696x; 2.0081x over previous
import functools

import jax
import jax.numpy as jnp
from jax import lax
from jax.experimental import pallas as pl
from jax.experimental.pallas import tpu as pltpu
from jax.experimental.pallas import tpu_sc as plsc

_B, _S, _D = 4, 8192, 1024
_NW = 32
_SW = _S // _NW
_RC = 8
_NCH = _SW // _RC
_KV = _D // 16


def _sc_body(x_hbm, t_hbm, o_hbm, xA, xB, tA, tB, siA, siB, stA, stB,
             soA, soB):
    cid = lax.axis_index("c")
    sid = lax.axis_index("s")
    wid = sid * 2 + cid
    s0 = wid * _SW

    def in_copy(c, xbuf, b, sem):
        return pltpu.make_async_copy(
            x_hbm.at[b, pl.ds(s0 + c * _RC, _RC), :], xbuf.at[b], sem)

    def out_copy(c, xbuf, b, sem):
        return pltpu.make_async_copy(
            xbuf.at[b], o_hbm.at[b, pl.ds(s0 + c * _RC, _RC), :], sem)

    def t_copy(c, tbuf, sem):
        return pltpu.make_async_copy(
            t_hbm.at[pl.ds(s0 + c * _RC, _RC), :], tbuf, sem)

    def start_chunk(c, xbuf, tbuf, sem_in, sem_t):
        t_copy(c, tbuf, sem_t).start()
        for b in range(_B):
            in_copy(c, xbuf, b, sem_in).start()

    def wait_chunk_in(c, xbuf, tbuf, sem_in, sem_t):
        t_copy(c, tbuf, sem_t).wait()
        for b in range(_B):
            in_copy(c, xbuf, b, sem_in).wait()

    def compute(xbuf, tbuf):
        @plsc.parallel_loop(0, _RC, unroll=2)
        def _(i):
            for k in range(_KV):
                sl = pl.ds(k * 16, 16)
                tv = tbuf[i, sl]
                for b in range(_B):
                    xbuf[b, i, sl] = xbuf[b, i, sl] + tv

    start_chunk(0, xA, tA, siA, stA)
    start_chunk(1, xB, tB, siB, stB)

    def iteration(g, carry):
        c0 = 2 * g
        c1 = c0 + 1
        wait_chunk_in(c0, xA, tA, siA, stA)
        compute(xA, tA)
        for b in range(_B):
            out_copy(c0, xA, b, soA).start()
        wait_chunk_in(c1, xB, tB, siB, stB)
        compute(xB, tB)
        for b in range(_B):
            out_copy(c1, xB, b, soB).start()
        for b in range(_B):
            out_copy(c0, xA, b, soA).wait()

        @pl.when(c0 + 2 < _NCH)
        def _():
            start_chunk(c0 + 2, xA, tA, siA, stA)

        for b in range(_B):
            out_copy(c1, xB, b, soB).wait()

        @pl.when(c1 + 2 < _NCH)
        def _():
            start_chunk(c1 + 2, xB, tB, siB, stB)

        return carry

    lax.fori_loop(0, _NCH // 2, iteration, 0)


def kernel(x, table):
    mesh = plsc.VectorSubcoreMesh(core_axis_name="c", subcore_axis_name="s")
    k = functools.partial(
        pl.kernel,
        mesh=mesh,
        out_type=jax.ShapeDtypeStruct((_B, _S, _D), jnp.float32),
        scratch_types=[
            pltpu.VMEM((_B, _RC, _D), jnp.float32),
            pltpu.VMEM((_B, _RC, _D), jnp.float32),
            pltpu.VMEM((_RC, _D), jnp.float32),
            pltpu.VMEM((_RC, _D), jnp.float32),
            pltpu.SemaphoreType.DMA,
            pltpu.SemaphoreType.DMA,
            pltpu.SemaphoreType.DMA,
            pltpu.SemaphoreType.DMA,
            pltpu.SemaphoreType.DMA,
            pltpu.SemaphoreType.DMA,
        ],
    )(_sc_body)
    return k(x, table)


# speedup vs baseline: 5.1251x; 1.5210x over previous
import functools

import jax
import jax.numpy as jnp
from jax import lax
from jax.experimental import pallas as pl
from jax.experimental.pallas import tpu as pltpu
from jax.experimental.pallas import tpu_sc as plsc

_B, _S, _D = 4, 8192, 1024
_NW = 32
_SW = _S // _NW
_RC = 8
_NCH = _SW // _RC
_KV = _D // 16


def _sc_body(x_hbm, t_hbm, o_hbm, xA, xB, tA, tB, siA, siB, stA, stB,
             soA, soB):
    cid = lax.axis_index("c")
    sid = lax.axis_index("s")
    wid = sid * 2 + cid
    s0 = wid * _SW

    def in_copy(c, xbuf, b, sem):
        return pltpu.make_async_copy(
            x_hbm.at[b, pl.ds(s0 + c * _RC, _RC), :], xbuf.at[b], sem)

    def out_copy(c, xbuf, b, sem):
        return pltpu.make_async_copy(
            xbuf.at[b], o_hbm.at[b, pl.ds(s0 + c * _RC, _RC), :], sem)

    def t_copy(c, tbuf, sem):
        return pltpu.make_async_copy(
            t_hbm.at[pl.ds(s0 + c * _RC, _RC), :], tbuf, sem)

    def start_chunk(c, xbuf, tbuf, sem_in, sem_t):
        t_copy(c, tbuf, sem_t).start()
        for b in range(_B):
            in_copy(c, xbuf, b, sem_in).start()

    def wait_chunk_in(c, xbuf, tbuf, sem_in, sem_t):
        t_copy(c, tbuf, sem_t).wait()
        for b in range(_B):
            in_copy(c, xbuf, b, sem_in).wait()

    def compute(xbuf, tbuf):
        @plsc.parallel_loop(0, _KV, unroll=2)
        def _(k):
            sl = pl.ds(k * 16, 16)
            for i in range(_RC):
                tv = tbuf[i, sl]
                for b in range(_B):
                    xbuf[b, i, sl] = xbuf[b, i, sl] + tv

    start_chunk(0, xA, tA, siA, stA)
    start_chunk(1, xB, tB, siB, stB)

    def iteration(g, carry):
        c0 = 2 * g
        c1 = c0 + 1
        wait_chunk_in(c0, xA, tA, siA, stA)
        compute(xA, tA)
        for b in range(_B):
            out_copy(c0, xA, b, soA).start()
        wait_chunk_in(c1, xB, tB, siB, stB)
        compute(xB, tB)
        for b in range(_B):
            out_copy(c1, xB, b, soB).start()
        for b in range(_B):
            out_copy(c0, xA, b, soA).wait()

        @pl.when(c0 + 2 < _NCH)
        def _():
            start_chunk(c0 + 2, xA, tA, siA, stA)

        for b in range(_B):
            out_copy(c1, xB, b, soB).wait()

        @pl.when(c1 + 2 < _NCH)
        def _():
            start_chunk(c1 + 2, xB, tB, siB, stB)

        return carry

    lax.fori_loop(0, _NCH // 2, iteration, 0)


def kernel(x, table):
    mesh = plsc.VectorSubcoreMesh(core_axis_name="c", subcore_axis_name="s")
    k = functools.partial(
        pl.kernel,
        mesh=mesh,
        out_type=jax.ShapeDtypeStruct((_B, _S, _D), jnp.float32),
        scratch_types=[
            pltpu.VMEM((_B, _RC, _D), jnp.float32),
            pltpu.VMEM((_B, _RC, _D), jnp.float32),
            pltpu.VMEM((_RC, _D), jnp.float32),
            pltpu.VMEM((_RC, _D), jnp.float32),
            pltpu.SemaphoreType.DMA,
            pltpu.SemaphoreType.DMA,
            pltpu.SemaphoreType.DMA,
            pltpu.SemaphoreType.DMA,
            pltpu.SemaphoreType.DMA,
            pltpu.SemaphoreType.DMA,
        ],
    )(_sc_body)
    return k(x, table)
